# baseline (speedup 1.0000x reference)
import functools

import jax
import jax.numpy as jnp
from jax.experimental import pallas as pl

B, N, D, H, K = 8, 2048, 128, 8, 12
DH = D // H


def _combine_body(out_ref, wavg_ref, lin_w_ref, lin_b_ref, y_ref):
    wavg = wavg_ref[...]
    lw = lin_w_ref[...]
    lb = lin_b_ref[0, 0]
    logits = jnp.sum(wavg * lw[None, :, :], axis=-1) + lb
    x_s = jax.nn.sigmoid(logits)
    a = x_s / jnp.sum(x_s, axis=1, keepdims=True)
    y_ref[...] = jnp.sum(a[:, :, None] * out_ref[...], axis=1)


def _combine(out, wavg, lin_w, lin_b):
    BN = out.shape[0]
    T = 512
    grid = (BN // T,)
    return pl.pallas_call(
        _combine_body,
        grid=grid,
        in_specs=[
            pl.BlockSpec((T, K, D), lambda i: (i, 0, 0)),
            pl.BlockSpec((T, K, K), lambda i: (i, 0, 0)),
            pl.BlockSpec((1, K), lambda i: (0, 0)),
            pl.BlockSpec((1, 1), lambda i: (0, 0)),
        ],
        out_specs=pl.BlockSpec((T, D), lambda i: (i, 0)),
        out_shape=jax.ShapeDtypeStruct((BN, D), jnp.float32),
    )(out, wavg, lin_w, lin_b.reshape(1, 1))


def kernel(x, in_proj_w, in_proj_b, out_proj_w, out_proj_b, lin_w, lin_b):
    sq = jnp.sum(x * x, axis=-1)
    d2 = sq[:, :, None] + sq[:, None, :] - 2.0 * (x @ x.transpose(0, 2, 1))
    _, idx = jax.lax.top_k(-d2, K)
    gathered = x[jnp.arange(B)[:, None, None], idx, :]
    xr = gathered.reshape(B * N, K, D)
    qkv = xr @ in_proj_w.T + in_proj_b
    q, k, v = jnp.split(qkv, 3, axis=-1)

    def sp(t):
        return t.reshape(B * N, K, H, DH).transpose(0, 2, 1, 3)

    q, k, v = sp(q), sp(k), sp(v)
    scores = (q @ k.transpose(0, 1, 3, 2)) / jnp.sqrt(jnp.float32(DH))
    w = jax.nn.softmax(scores, axis=-1)
    o = (w @ v).transpose(0, 2, 1, 3).reshape(B * N, K, D)
    out = o @ out_proj_w.T + out_proj_b
    wavg = jnp.mean(w, axis=1)
    y = _combine(out, wavg, lin_w, lin_b)
    return y.reshape(B, N, D)


# speedup vs baseline: 1.7433x; 1.7433x over previous
import functools

import jax
import jax.numpy as jnp
from jax.experimental import pallas as pl

B, N, D, H, K = 8, 2048, 128, 8, 12
DH = D // H
T = 128
NT = (B * N) // T


def _attn_body(xg_ref, w_in_ref, b_in_ref, w_out_ref, b_out_ref, lw_ref,
               lb_ref, y_ref):
    W = w_in_ref[...]
    b_in = b_in_ref[...]
    scale = 1.0 / jnp.sqrt(jnp.float32(DH))

    qkvT = []
    for i in range(K):
        Xi = xg_ref[0, i * T:(i + 1) * T, :]
        zi = jax.lax.dot_general(W, Xi, (((1,), (1,)), ((), ())),
                                 preferred_element_type=jnp.float32)
        qkvT.append(zi + b_in.reshape(3 * D, 1))

    wavg = [jnp.zeros((K, T), jnp.float32) for _ in range(K)]
    o_acc = [[] for _ in range(K)]

    for h in range(H):
        qs = [qkvT[i][h * DH:(h + 1) * DH, :] for i in range(K)]
        ks = [qkvT[j][D + h * DH:D + (h + 1) * DH, :] for j in range(K)]
        vs = [qkvT[j][2 * D + h * DH:2 * D + (h + 1) * DH, :] for j in range(K)]
        for i in range(K):
            s_rows = []
            for j in range(K):
                s_rows.append(jnp.sum(qs[i] * ks[j], axis=0) * scale)
            S = jnp.stack(s_rows, axis=0)
            S = S - jnp.max(S, axis=0, keepdims=True)
            E = jnp.exp(S)
            Wij = E / jnp.sum(E, axis=0, keepdims=True)
            wavg[i] = wavg[i] + Wij
            oh = jnp.zeros((DH, T), jnp.float32)
            for j in range(K):
                oh = oh + Wij[j][None, :] * vs[j]
            o_acc[i].append(oh)

    lw = lw_ref[...]
    lb = lb_ref[0, 0]
    xs_rows = []
    for i in range(K):
        wavg_i = wavg[i] * (1.0 / H)
        logit = jnp.sum(wavg_i * lw.reshape(K, 1), axis=0) + lb
        xs_rows.append(jax.nn.sigmoid(logit))
    A = jnp.stack(xs_rows, axis=0)
    A = A / jnp.sum(A, axis=0, keepdims=True)

    yt = jnp.zeros((D, T), jnp.float32)
    for i in range(K):
        o_i = jnp.concatenate(o_acc[i], axis=0)
        yt = yt + A[i][None, :] * o_i
    yT = jax.lax.dot_general(w_out_ref[...], yt, (((1,), (0,)), ((), ())),
                             preferred_element_type=jnp.float32)
    yT = yT + b_out_ref[...].reshape(D, 1)
    y_ref[0] = yT.T


def _attention(xg, w_in, b_in, w_out, b_out, lin_w, lin_b):
    return pl.pallas_call(
        _attn_body,
        grid=(NT,),
        in_specs=[
            pl.BlockSpec((1, K * T, D), lambda i: (i, 0, 0)),
            pl.BlockSpec((3 * D, D), lambda i: (0, 0)),
            pl.BlockSpec((1, 3 * D), lambda i: (0, 0)),
            pl.BlockSpec((D, D), lambda i: (0, 0)),
            pl.BlockSpec((1, D), lambda i: (0, 0)),
            pl.BlockSpec((1, K), lambda i: (0, 0)),
            pl.BlockSpec((1, 1), lambda i: (0, 0)),
        ],
        out_specs=pl.BlockSpec((1, T, D), lambda i: (i, 0, 0)),
        out_shape=jax.ShapeDtypeStruct((NT, T, D), jnp.float32),
    )(xg, w_in, b_in.reshape(1, 3 * D), w_out, b_out.reshape(1, D),
      lin_w, lin_b.reshape(1, 1))


def kernel(x, in_proj_w, in_proj_b, out_proj_w, out_proj_b, lin_w, lin_b):
    sq = jnp.sum(x * x, axis=-1)
    d2 = sq[:, :, None] + sq[:, None, :] - 2.0 * (x @ x.transpose(0, 2, 1))
    _, idx = jax.lax.top_k(-d2, K)
    gid = jnp.arange(B)[:, None, None] * N + idx
    gid = gid.reshape(NT, T, K).transpose(0, 2, 1)
    x2d = x.reshape(B * N, D)
    xg = x2d[gid.reshape(-1), :].reshape(NT, K * T, D)
    y = _attention(xg, in_proj_w, in_proj_b, out_proj_w, out_proj_b,
                   lin_w, lin_b)
    return y.reshape(B, N, D)


# speedup vs baseline: 7.7816x; 4.4637x over previous
import functools

import jax
import jax.numpy as jnp
from jax.experimental import pallas as pl

B, N, D, H, K = 8, 2048, 128, 8, 12
DH = D // H
T = 128
NT = (B * N) // T


def _attn_body(xg_ref, w_in_ref, b_in_ref, w_out_ref, b_out_ref, lw_ref,
               lb_ref, y_ref):
    W = w_in_ref[...]
    b_in = b_in_ref[...]
    scale = 1.0 / jnp.sqrt(jnp.float32(DH))

    qkvT = []
    for i in range(K):
        Xi = xg_ref[0, i * T:(i + 1) * T, :]
        zi = jax.lax.dot_general(W, Xi, (((1,), (1,)), ((), ())),
                                 preferred_element_type=jnp.float32)
        qkvT.append(zi + b_in.reshape(3 * D, 1))

    wavg = [jnp.zeros((K, T), jnp.float32) for _ in range(K)]
    o_acc = [[] for _ in range(K)]

    for h in range(H):
        qs = [qkvT[i][h * DH:(h + 1) * DH, :] for i in range(K)]
        ks = [qkvT[j][D + h * DH:D + (h + 1) * DH, :] for j in range(K)]
        vs = [qkvT[j][2 * D + h * DH:2 * D + (h + 1) * DH, :] for j in range(K)]
        for i in range(K):
            s_rows = []
            for j in range(K):
                s_rows.append(jnp.sum(qs[i] * ks[j], axis=0) * scale)
            S = jnp.stack(s_rows, axis=0)
            S = S - jnp.max(S, axis=0, keepdims=True)
            E = jnp.exp(S)
            Wij = E / jnp.sum(E, axis=0, keepdims=True)
            wavg[i] = wavg[i] + Wij
            oh = jnp.zeros((DH, T), jnp.float32)
            for j in range(K):
                oh = oh + Wij[j][None, :] * vs[j]
            o_acc[i].append(oh)

    lw = lw_ref[...]
    lb = lb_ref[0, 0]
    xs_rows = []
    for i in range(K):
        wavg_i = wavg[i] * (1.0 / H)
        logit = jnp.sum(wavg_i * lw.reshape(K, 1), axis=0) + lb
        xs_rows.append(jax.nn.sigmoid(logit))
    A = jnp.stack(xs_rows, axis=0)
    A = A / jnp.sum(A, axis=0, keepdims=True)

    yt = jnp.zeros((D, T), jnp.float32)
    for i in range(K):
        o_i = jnp.concatenate(o_acc[i], axis=0)
        yt = yt + A[i][None, :] * o_i
    yT = jax.lax.dot_general(w_out_ref[...], yt, (((1,), (0,)), ((), ())),
                             preferred_element_type=jnp.float32)
    yT = yT + b_out_ref[...].reshape(D, 1)
    y_ref[0] = yT.T


def _attention(xg, w_in, b_in, w_out, b_out, lin_w, lin_b):
    return pl.pallas_call(
        _attn_body,
        grid=(NT,),
        in_specs=[
            pl.BlockSpec((1, K * T, D), lambda i: (i, 0, 0)),
            pl.BlockSpec((3 * D, D), lambda i: (0, 0)),
            pl.BlockSpec((1, 3 * D), lambda i: (0, 0)),
            pl.BlockSpec((D, D), lambda i: (0, 0)),
            pl.BlockSpec((1, D), lambda i: (0, 0)),
            pl.BlockSpec((1, K), lambda i: (0, 0)),
            pl.BlockSpec((1, 1), lambda i: (0, 0)),
        ],
        out_specs=pl.BlockSpec((1, T, D), lambda i: (i, 0, 0)),
        out_shape=jax.ShapeDtypeStruct((NT, T, D), jnp.float32),
    )(xg, w_in, b_in.reshape(1, 3 * D), w_out, b_out.reshape(1, D),
      lin_w, lin_b.reshape(1, 1))


R = 256
_BIG = 3.0e38


def _knn_body(x_all_ref, xr_ref, sq_ref, idx_ref):
    xa = x_all_ref[0]
    xr = xr_ref[0]
    sq = sq_ref[0]
    dots = jax.lax.dot_general(xr, xa, (((1,), (1,)), ((), ())),
                               preferred_element_type=jnp.float32)
    cur = sq - 2.0 * dots
    lane = jax.lax.broadcasted_iota(jnp.int32, (R, N), 1)
    cols = []
    for _ in range(K):
        m = jnp.min(cur, axis=1, keepdims=True)
        hit = cur == m
        amin = jnp.min(jnp.where(hit, lane, N), axis=1, keepdims=True)
        cols.append(amin)
        cur = jnp.where(lane == amin, _BIG, cur)
    idx_ref[0] = jnp.concatenate(cols, axis=1)


def _knn(x):
    sq = jnp.sum(x * x, axis=-1).reshape(B, 1, N)
    return pl.pallas_call(
        _knn_body,
        grid=(B, N // R),
        in_specs=[
            pl.BlockSpec((1, N, D), lambda b, r: (b, 0, 0)),
            pl.BlockSpec((1, R, D), lambda b, r: (b, r, 0)),
            pl.BlockSpec((1, 1, N), lambda b, r: (b, 0, 0)),
        ],
        out_specs=pl.BlockSpec((1, R, K), lambda b, r: (b, r, 0)),
        out_shape=jax.ShapeDtypeStruct((B, N, K), jnp.int32),
    )(x, x, sq)


def kernel(x, in_proj_w, in_proj_b, out_proj_w, out_proj_b, lin_w, lin_b):
    idx = _knn(x)
    gid = jnp.arange(B)[:, None, None] * N + idx
    gid = gid.reshape(NT, T, K).transpose(0, 2, 1)
    x2d = x.reshape(B * N, D)
    xg = x2d[gid.reshape(-1), :].reshape(NT, K * T, D)
    y = _attention(xg, in_proj_w, in_proj_b, out_proj_w, out_proj_b,
                   lin_w, lin_b)
    return y.reshape(B, N, D)


# speedup vs baseline: 11.7395x; 1.5086x over previous
import functools

import jax
import jax.numpy as jnp
from jax import lax
from jax.experimental import pallas as pl
from jax.experimental.pallas import tpu as pltpu
from jax.experimental.pallas import tpu_sc as plsc

B, N, D, H, K = 8, 2048, 128, 8, 12
DH = D // H
T = 128
NT = (B * N) // T


def _attn_body(xg_ref, w_in_ref, b_in_ref, w_out_ref, b_out_ref, lw_ref,
               lb_ref, y_ref):
    W = w_in_ref[...]
    b_in = b_in_ref[...]
    scale = 1.0 / jnp.sqrt(jnp.float32(DH))

    qkvT = []
    for i in range(K):
        Xi = xg_ref[0, i * T:(i + 1) * T, :]
        zi = jax.lax.dot_general(W, Xi, (((1,), (1,)), ((), ())),
                                 preferred_element_type=jnp.float32)
        qkvT.append(zi + b_in.reshape(3 * D, 1))

    wavg = [jnp.zeros((K, T), jnp.float32) for _ in range(K)]
    o_acc = [[] for _ in range(K)]

    for h in range(H):
        qs = [qkvT[i][h * DH:(h + 1) * DH, :] for i in range(K)]
        ks = [qkvT[j][D + h * DH:D + (h + 1) * DH, :] for j in range(K)]
        vs = [qkvT[j][2 * D + h * DH:2 * D + (h + 1) * DH, :] for j in range(K)]
        for i in range(K):
            s_rows = []
            for j in range(K):
                s_rows.append(jnp.sum(qs[i] * ks[j], axis=0) * scale)
            S = jnp.stack(s_rows, axis=0)
            S = S - jnp.max(S, axis=0, keepdims=True)
            E = jnp.exp(S)
            Wij = E / jnp.sum(E, axis=0, keepdims=True)
            wavg[i] = wavg[i] + Wij
            oh = jnp.zeros((DH, T), jnp.float32)
            for j in range(K):
                oh = oh + Wij[j][None, :] * vs[j]
            o_acc[i].append(oh)

    lw = lw_ref[...]
    lb = lb_ref[0, 0]
    xs_rows = []
    for i in range(K):
        wavg_i = wavg[i] * (1.0 / H)
        logit = jnp.sum(wavg_i * lw.reshape(K, 1), axis=0) + lb
        xs_rows.append(jax.nn.sigmoid(logit))
    A = jnp.stack(xs_rows, axis=0)
    A = A / jnp.sum(A, axis=0, keepdims=True)

    yt = jnp.zeros((D, T), jnp.float32)
    for i in range(K):
        o_i = jnp.concatenate(o_acc[i], axis=0)
        yt = yt + A[i][None, :] * o_i
    yT = jax.lax.dot_general(w_out_ref[...], yt, (((1,), (0,)), ((), ())),
                             preferred_element_type=jnp.float32)
    yT = yT + b_out_ref[...].reshape(D, 1)
    y_ref[0] = yT.T


def _attention(xg, w_in, b_in, w_out, b_out, lin_w, lin_b):
    return pl.pallas_call(
        _attn_body,
        grid=(NT,),
        in_specs=[
            pl.BlockSpec((1, K * T, D), lambda i: (i, 0, 0)),
            pl.BlockSpec((3 * D, D), lambda i: (0, 0)),
            pl.BlockSpec((1, 3 * D), lambda i: (0, 0)),
            pl.BlockSpec((D, D), lambda i: (0, 0)),
            pl.BlockSpec((1, D), lambda i: (0, 0)),
            pl.BlockSpec((1, K), lambda i: (0, 0)),
            pl.BlockSpec((1, 1), lambda i: (0, 0)),
        ],
        out_specs=pl.BlockSpec((1, T, D), lambda i: (i, 0, 0)),
        out_shape=jax.ShapeDtypeStruct((NT, T, D), jnp.float32),
    )(xg, w_in, b_in.reshape(1, 3 * D), w_out, b_out.reshape(1, D),
      lin_w, lin_b.reshape(1, 1))


R = 256
_BIG = 3.0e38


def _knn_body(x_all_ref, xr_ref, sq_ref, idx_ref):
    xa = x_all_ref[0]
    xr = xr_ref[0]
    sq = sq_ref[0]
    dots = jax.lax.dot_general(xr, xa, (((1,), (1,)), ((), ())),
                               preferred_element_type=jnp.float32)
    cur = sq - 2.0 * dots
    lane = jax.lax.broadcasted_iota(jnp.int32, (R, N), 1)
    cols = []
    for _ in range(K):
        m = jnp.min(cur, axis=1, keepdims=True)
        hit = cur == m
        amin = jnp.min(jnp.where(hit, lane, N), axis=1, keepdims=True)
        cols.append(amin)
        cur = jnp.where(lane == amin, _BIG, cur)
    idx_ref[0] = jnp.concatenate(cols, axis=1)


def _knn(x):
    sq = jnp.sum(x * x, axis=-1).reshape(B, 1, N)
    return pl.pallas_call(
        _knn_body,
        grid=(B, N // R),
        in_specs=[
            pl.BlockSpec((1, N, D), lambda b, r: (b, 0, 0)),
            pl.BlockSpec((1, R, D), lambda b, r: (b, r, 0)),
            pl.BlockSpec((1, 1, N), lambda b, r: (b, 0, 0)),
        ],
        out_specs=pl.BlockSpec((1, R, K), lambda b, r: (b, r, 0)),
        out_shape=jax.ShapeDtypeStruct((B, N, K), jnp.int32),
    )(x, x, sq)


_NW = 32
_RPW = (B * N * K) // _NW
_CH = 256
_NCH = _RPW // _CH


def _gather_sc(x2d, gidp):
    mesh = plsc.VectorSubcoreMesh(core_axis_name="c", subcore_axis_name="s")

    @functools.partial(
        pl.kernel, mesh=mesh,
        out_type=jax.ShapeDtypeStruct((B * N * K, D), jnp.float32),
        scratch_types=[
            pltpu.VMEM((_CH,), jnp.int32),
            pltpu.VMEM((_CH,), jnp.int32),
            pltpu.VMEM((_CH, D), jnp.float32),
            pltpu.VMEM((_CH, D), jnp.float32),
            pltpu.SemaphoreType.DMA,
            pltpu.SemaphoreType.DMA,
        ],
    )
    def k(x_hbm, idx_hbm, out_hbm, idx0, idx1, buf0, buf1, sem0, sem1):
        wid = lax.axis_index("s") * 2 + lax.axis_index("c")
        base = wid * _RPW
        idxs = (idx0, idx1)
        bufs = (buf0, buf1)
        sems = (sem0, sem1)
        cps = [None, None]
        pltpu.sync_copy(idx_hbm.at[wid, 0], idx0)
        cps[0] = pltpu.async_copy(x_hbm.at[idx0], buf0, sem0)
        for c in range(_NCH):
            if c + 1 < _NCH:
                nb = (c + 1) % 2
                pltpu.sync_copy(idx_hbm.at[wid, c + 1], idxs[nb])
                cps[nb] = pltpu.async_copy(x_hbm.at[idxs[nb]], bufs[nb],
                                           sems[nb])
            cps[c % 2].wait()
            pltpu.sync_copy(bufs[c % 2],
                            out_hbm.at[pl.ds(base + c * _CH, _CH)])

    return k(x2d, gidp)


def kernel(x, in_proj_w, in_proj_b, out_proj_w, out_proj_b, lin_w, lin_b):
    idx = _knn(x)
    gid = jnp.arange(B)[:, None, None] * N + idx
    gid = gid.reshape(NT, T, K).transpose(0, 2, 1)
    x2d = x.reshape(B * N, D)
    xg = _gather_sc(x2d, gid.reshape(_NW, _NCH, _CH)).reshape(NT, K * T, D)
    y = _attention(xg, in_proj_w, in_proj_b, out_proj_w, out_proj_b,
                   lin_w, lin_b)
    return y.reshape(B, N, D)


# speedup vs baseline: 12.1960x; 1.0389x over previous
import functools

import jax
import jax.numpy as jnp
from jax import lax
from jax.experimental import pallas as pl
from jax.experimental.pallas import tpu as pltpu
from jax.experimental.pallas import tpu_sc as plsc

B, N, D, H, K = 8, 2048, 128, 8, 12
DH = D // H
T = 128
NT = (B * N) // T


def _attn_body(xg_ref, w_in_ref, b_in_ref, w_out_ref, b_out_ref, lw_ref,
               lb_ref, y_ref):
    W = w_in_ref[...]
    b_in = b_in_ref[...]
    scale = 1.0 / jnp.sqrt(jnp.float32(DH))

    qkvT = []
    for i in range(K):
        Xi = xg_ref[0, i * T:(i + 1) * T, :]
        zi = jax.lax.dot_general(W, Xi, (((1,), (1,)), ((), ())),
                                 preferred_element_type=jnp.float32)
        qkvT.append(zi + b_in.reshape(3 * D, 1))

    wavg = [jnp.zeros((K, T), jnp.float32) for _ in range(K)]
    o_acc = [[] for _ in range(K)]

    for h in range(H):
        qs = [qkvT[i][h * DH:(h + 1) * DH, :] for i in range(K)]
        ks = [qkvT[j][D + h * DH:D + (h + 1) * DH, :] for j in range(K)]
        vs = [qkvT[j][2 * D + h * DH:2 * D + (h + 1) * DH, :] for j in range(K)]
        for i in range(K):
            s_rows = []
            for j in range(K):
                s_rows.append(jnp.sum(qs[i] * ks[j], axis=0) * scale)
            S = jnp.stack(s_rows, axis=0)
            S = S - jnp.max(S, axis=0, keepdims=True)
            E = jnp.exp(S)
            Wij = E / jnp.sum(E, axis=0, keepdims=True)
            wavg[i] = wavg[i] + Wij
            oh = jnp.zeros((DH, T), jnp.float32)
            for j in range(K):
                oh = oh + Wij[j][None, :] * vs[j]
            o_acc[i].append(oh)

    lw = lw_ref[...]
    lb = lb_ref[0, 0]
    xs_rows = []
    for i in range(K):
        wavg_i = wavg[i] * (1.0 / H)
        logit = jnp.sum(wavg_i * lw.reshape(K, 1), axis=0) + lb
        xs_rows.append(jax.nn.sigmoid(logit))
    A = jnp.stack(xs_rows, axis=0)
    A = A / jnp.sum(A, axis=0, keepdims=True)

    yt = jnp.zeros((D, T), jnp.float32)
    for i in range(K):
        o_i = jnp.concatenate(o_acc[i], axis=0)
        yt = yt + A[i][None, :] * o_i
    yT = jax.lax.dot_general(w_out_ref[...], yt, (((1,), (0,)), ((), ())),
                             preferred_element_type=jnp.float32)
    yT = yT + b_out_ref[...].reshape(D, 1)
    y_ref[0] = yT.T


def _attention(xg, w_in, b_in, w_out, b_out, lin_w, lin_b):
    nt = xg.shape[0]
    return pl.pallas_call(
        _attn_body,
        grid=(nt,),
        in_specs=[
            pl.BlockSpec((1, K * T, D), lambda i: (i, 0, 0)),
            pl.BlockSpec((3 * D, D), lambda i: (0, 0)),
            pl.BlockSpec((1, 3 * D), lambda i: (0, 0)),
            pl.BlockSpec((D, D), lambda i: (0, 0)),
            pl.BlockSpec((1, D), lambda i: (0, 0)),
            pl.BlockSpec((1, K), lambda i: (0, 0)),
            pl.BlockSpec((1, 1), lambda i: (0, 0)),
        ],
        out_specs=pl.BlockSpec((1, T, D), lambda i: (i, 0, 0)),
        out_shape=jax.ShapeDtypeStruct((nt, T, D), jnp.float32),
    )(xg, w_in, b_in.reshape(1, 3 * D), w_out, b_out.reshape(1, D),
      lin_w, lin_b.reshape(1, 1))


R = 256
_BIG = 3.0e38


def _knn_body(x_all_ref, xr_ref, sq_ref, idx_ref):
    xa = x_all_ref[0]
    xr = xr_ref[0]
    sq = sq_ref[0]
    dots = jax.lax.dot_general(xr, xa, (((1,), (1,)), ((), ())),
                               preferred_element_type=jnp.float32)
    cur = sq - 2.0 * dots
    lane = jax.lax.broadcasted_iota(jnp.int32, (R, N), 1)
    cols = []
    for _ in range(K):
        m = jnp.min(cur, axis=1, keepdims=True)
        hit = cur == m
        amin = jnp.min(jnp.where(hit, lane, N), axis=1, keepdims=True)
        cols.append(amin)
        cur = jnp.where(lane == amin, _BIG, cur)
    idx_ref[0] = jnp.concatenate(cols, axis=1)


def _knn(x):
    nb = x.shape[0]
    sq = jnp.sum(x * x, axis=-1).reshape(nb, 1, N)
    return pl.pallas_call(
        _knn_body,
        grid=(nb, N // R),
        in_specs=[
            pl.BlockSpec((1, N, D), lambda b, r: (b, 0, 0)),
            pl.BlockSpec((1, R, D), lambda b, r: (b, r, 0)),
            pl.BlockSpec((1, 1, N), lambda b, r: (b, 0, 0)),
        ],
        out_specs=pl.BlockSpec((1, R, K), lambda b, r: (b, r, 0)),
        out_shape=jax.ShapeDtypeStruct((nb, N, K), jnp.int32),
    )(x, x, sq)


_NW = 32
_RPW = (B * N * K) // _NW
_CH = 256
_NCH = _RPW // _CH


def _gather_sc(x2d, gidp):
    rpw = gidp.shape[2]
    mesh = plsc.VectorSubcoreMesh(core_axis_name="c", subcore_axis_name="s")

    @functools.partial(
        pl.kernel, mesh=mesh,
        out_type=jax.ShapeDtypeStruct((_NW * rpw, D), jnp.float32),
        scratch_types=[
            pltpu.VMEM((rpw,), jnp.int32),
            pltpu.VMEM((rpw, D), jnp.float32),
            pltpu.SemaphoreType.DMA,
        ],
    )
    def k(x_hbm, idx_hbm, out_hbm, idx_v, buf, sem):
        wid = lax.axis_index("s") * 2 + lax.axis_index("c")
        pltpu.sync_copy(idx_hbm.at[wid, 0], idx_v)
        pltpu.async_copy(x_hbm.at[idx_v], buf, sem).wait()
        pltpu.sync_copy(buf, out_hbm.at[pl.ds(wid * rpw, rpw)])

    return k(x2d, gidp)


def kernel(x, in_proj_w, in_proj_b, out_proj_w, out_proj_b, lin_w, lin_b):
    ntb = N // T
    rpw = (N * K) // _NW
    ys = []
    for b in range(B):
        xb = x[b:b + 1]
        idx_b = _knn(xb)
        gid = idx_b.reshape(ntb, T, K).transpose(0, 2, 1)
        xg = _gather_sc(x[b], gid.reshape(_NW, 1, rpw))
        y = _attention(xg.reshape(ntb, K * T, D), in_proj_w, in_proj_b,
                       out_proj_w, out_proj_b, lin_w, lin_b)
        ys.append(y.reshape(N, D))
    return jnp.stack(ys, axis=0)
